# baseline (speedup 1.0000x reference)
import functools

import jax
import jax.numpy as jnp
from jax.experimental import pallas as pl
from jax.experimental.pallas import tpu as pltpu

_LB = 4000


def _nn_kernel(u_ref, l_ref, out_ref, acc_ref, *, nblocks):
    i = pl.program_id(0)

    @pl.when(i == 0)
    def _init():
        acc_ref[:] = jnp.full_like(acc_ref, jnp.inf)

    l = l_ref[:]
    dot = jax.lax.dot_general(
        u_ref[:], l, (((1,), (1,)), ((), ())),
        preferred_element_type=jnp.float32)
    ones = jnp.ones((1, l.shape[1]), dtype=l.dtype)
    l2 = jax.lax.dot_general(
        ones, l * l, (((1,), (1,)), ((), ())),
        preferred_element_type=jnp.float32)
    m = jnp.min(dot + l2, axis=1, keepdims=True)
    acc_ref[:] = jnp.minimum(acc_ref[:], m)

    @pl.when(i == nblocks - 1)
    def _finish():
        u = u_ref[:].astype(jnp.float32)
        u2 = 0.25 * jnp.sum(u * u, axis=1, keepdims=True)
        d = jnp.sqrt(jnp.maximum(acc_ref[:] + u2, 0.0))
        d = d - jnp.min(d)
        out_ref[:] = d / (jnp.max(d) + 1e-18)


def kernel(U_z, L_z):
    U = (U_z.reshape(U_z.shape[0], -1) * -2.0).astype(jnp.bfloat16)
    L = L_z.reshape(L_z.shape[0], -1).astype(jnp.bfloat16)
    n_u, k = U.shape
    n_l = L.shape[0]
    nblocks = n_l // _LB
    out = pl.pallas_call(
        functools.partial(_nn_kernel, nblocks=nblocks),
        grid=(nblocks,),
        in_specs=[
            pl.BlockSpec((n_u, k), lambda i: (0, 0)),
            pl.BlockSpec((_LB, k), lambda i: (i, 0)),
        ],
        out_specs=pl.BlockSpec((n_u, 1), lambda i: (0, 0)),
        out_shape=jax.ShapeDtypeStruct((n_u, 1), jnp.float32),
        scratch_shapes=[pltpu.VMEM((n_u, 1), jnp.float32)],
    )(U, L)
    return out.reshape(n_u)


# speedup vs baseline: 1.3253x; 1.3253x over previous
import functools

import jax
import jax.numpy as jnp
from jax.experimental import pallas as pl
from jax.experimental.pallas import tpu as pltpu

_LB = 4000


def _nn_kernel(u_ref, l_ref, out_ref, acc_ref, *, nblocks):
    i = pl.program_id(0)

    @pl.when(i == 0)
    def _init():
        acc_ref[:] = jnp.full_like(acc_ref, jnp.inf)

    l = l_ref[:]
    lb = l.astype(jnp.bfloat16)
    dot = jax.lax.dot_general(
        u_ref[:], lb, (((1,), (1,)), ((), ())),
        preferred_element_type=jnp.float32)
    ones = jnp.ones((1, l.shape[1]), dtype=jnp.float32)
    l2 = jax.lax.dot_general(
        ones, l * l, (((1,), (1,)), ((), ())),
        preferred_element_type=jnp.float32)
    m = jnp.min(dot + l2, axis=1, keepdims=True)
    acc_ref[:] = jnp.minimum(acc_ref[:], m)

    @pl.when(i == nblocks - 1)
    def _finish():
        u = u_ref[:].astype(jnp.float32)
        u2 = 0.25 * jnp.sum(u * u, axis=1, keepdims=True)
        d = jnp.sqrt(jnp.maximum(acc_ref[:] + u2, 0.0))
        d = d - jnp.min(d)
        out_ref[:] = d / (jnp.max(d) + 1e-18)


def kernel(U_z, L_z):
    U = (U_z.reshape(U_z.shape[0], -1) * -2.0).astype(jnp.bfloat16)
    L = L_z.reshape(L_z.shape[0], -1)
    n_u, k = U.shape
    n_l = L.shape[0]
    nblocks = n_l // _LB
    out = pl.pallas_call(
        functools.partial(_nn_kernel, nblocks=nblocks),
        grid=(nblocks,),
        in_specs=[
            pl.BlockSpec((n_u, k), lambda i: (0, 0)),
            pl.BlockSpec((_LB, k), lambda i: (i, 0)),
        ],
        out_specs=pl.BlockSpec((n_u, 1), lambda i: (0, 0)),
        out_shape=jax.ShapeDtypeStruct((n_u, 1), jnp.float32),
        scratch_shapes=[pltpu.VMEM((n_u, 1), jnp.float32)],
    )(U, L)
    return out.reshape(n_u)


# speedup vs baseline: 1.3733x; 1.0362x over previous
import functools

import jax
import jax.numpy as jnp
from jax.experimental import pallas as pl
from jax.experimental.pallas import tpu as pltpu

_LB = 5000


def _nn_kernel(u_ref, l_ref, out_ref, acc_ref, *, nblocks):
    i = pl.program_id(0)

    @pl.when(i == 0)
    def _init():
        acc_ref[:] = jnp.full_like(acc_ref, jnp.inf)

    l = l_ref[:]
    dot = jax.lax.dot_general(
        u_ref[:], l, (((1,), (1,)), ((), ())),
        preferred_element_type=jnp.float32)
    ones = jnp.ones((1, l.shape[1]), dtype=jnp.float32)
    l2 = jax.lax.dot_general(
        ones, l * l, (((1,), (1,)), ((), ())),
        preferred_element_type=jnp.float32)
    m = jnp.min(dot + l2, axis=1, keepdims=True)
    acc_ref[:] = jnp.minimum(acc_ref[:], m)

    @pl.when(i == nblocks - 1)
    def _finish():
        u = u_ref[:].astype(jnp.float32)
        u2 = 0.25 * jnp.sum(u * u, axis=1, keepdims=True)
        d = jnp.sqrt(jnp.maximum(acc_ref[:] + u2, 0.0))
        d = d - jnp.min(d)
        out_ref[:] = d / (jnp.max(d) + 1e-18)


def kernel(U_z, L_z):
    U = U_z.reshape(U_z.shape[0], -1) * -2.0
    L = L_z.reshape(L_z.shape[0], -1)
    n_u, k = U.shape
    n_l = L.shape[0]
    nblocks = n_l // _LB
    out = pl.pallas_call(
        functools.partial(_nn_kernel, nblocks=nblocks),
        grid=(nblocks,),
        in_specs=[
            pl.BlockSpec((n_u, k), lambda i: (0, 0)),
            pl.BlockSpec((_LB, k), lambda i: (i, 0)),
        ],
        out_specs=pl.BlockSpec((n_u, 1), lambda i: (0, 0)),
        out_shape=jax.ShapeDtypeStruct((n_u, 1), jnp.float32),
        scratch_shapes=[pltpu.VMEM((n_u, 1), jnp.float32)],
    )(U, L)
    return out.reshape(n_u)


# speedup vs baseline: 1.4303x; 1.0415x over previous
import functools

import jax
import jax.numpy as jnp
from jax.experimental import pallas as pl
from jax.experimental.pallas import tpu as pltpu

_LB = 10000


def _nn_kernel(u_ref, l_ref, out_ref, acc_ref, *, nblocks):
    i = pl.program_id(0)

    @pl.when(i == 0)
    def _init():
        acc_ref[:] = jnp.full_like(acc_ref, jnp.inf)

    l = l_ref[:]
    dot = jax.lax.dot_general(
        u_ref[:], l, (((1,), (1,)), ((), ())),
        preferred_element_type=jnp.float32)
    ones = jnp.ones((1, l.shape[1]), dtype=jnp.float32)
    l2 = jax.lax.dot_general(
        ones, l * l, (((1,), (1,)), ((), ())),
        preferred_element_type=jnp.float32)
    m = jnp.min(dot + l2, axis=1, keepdims=True)
    acc_ref[:] = jnp.minimum(acc_ref[:], m)

    @pl.when(i == nblocks - 1)
    def _finish():
        u = u_ref[:].astype(jnp.float32)
        u2 = 0.25 * jnp.sum(u * u, axis=1, keepdims=True)
        d = jnp.sqrt(jnp.maximum(acc_ref[:] + u2, 0.0))
        d = d - jnp.min(d)
        out_ref[:] = d / (jnp.max(d) + 1e-18)


def kernel(U_z, L_z):
    U = U_z.reshape(U_z.shape[0], -1) * -2.0
    L = L_z.reshape(L_z.shape[0], -1)
    n_u, k = U.shape
    n_l = L.shape[0]
    nblocks = n_l // _LB
    out = pl.pallas_call(
        functools.partial(_nn_kernel, nblocks=nblocks),
        grid=(nblocks,),
        in_specs=[
            pl.BlockSpec((n_u, k), lambda i: (0, 0)),
            pl.BlockSpec((_LB, k), lambda i: (i, 0)),
        ],
        out_specs=pl.BlockSpec((n_u, 1), lambda i: (0, 0)),
        out_shape=jax.ShapeDtypeStruct((n_u, 1), jnp.float32),
        scratch_shapes=[pltpu.VMEM((n_u, 1), jnp.float32)],
    )(U, L)
    return out.reshape(n_u)
